# baseline (speedup 1.0000x reference)
import functools

import jax
import jax.numpy as jnp
from jax import lax
from jax.experimental import pallas as pl
from jax.experimental.pallas import tpu as pltpu
from jax.experimental.pallas import tpu_sc as plsc

_VOCAB = 1000000
_H = 64
_N_CMDS = 1024
_CMD_LEN = 20
_OBS_LEN = 200

_NC, _NS = 2, 16
_NW = _NC * _NS
_CMDS_PER_W = _N_CMDS // _NW
_ROWS_PER_W = _CMDS_PER_W * _CMD_LEN
_IDX_CHUNK = 128
_N_CHUNKS = _ROWS_PER_W // _IDX_CHUNK
_OBS_PAD = 256

_GUMBEL = jax.random.gumbel(jax.random.key(123), (_N_CMDS, 1), jnp.float32)


@functools.lru_cache(maxsize=1)
def _sc_pool_kernel():
  mesh = plsc.VectorSubcoreMesh(
      core_axis_name="c", subcore_axis_name="s",
      num_cores=_NC, num_subcores=_NS,
  )

  @functools.partial(
      pl.kernel,
      out_type=[
          jax.ShapeDtypeStruct((_N_CMDS, _H), jnp.float32),
          jax.ShapeDtypeStruct((1, _H), jnp.float32),
      ],
      mesh=mesh,
      compiler_params=pltpu.CompilerParams(use_tc_tiling_on_sc=False),
      scratch_types=[
          pltpu.VMEM((_N_CHUNKS, _IDX_CHUNK), jnp.int32),
          pltpu.VMEM((_ROWS_PER_W, _H), jnp.float32),
          pltpu.VMEM((_CMDS_PER_W, _H), jnp.float32),
          pltpu.VMEM((2, _IDX_CHUNK), jnp.int32),
          pltpu.VMEM((_OBS_PAD, _H), jnp.float32),
          pltpu.VMEM((1, _H), jnp.float32),
          pltpu.SemaphoreType.DMA,
      ],
  )
  def sc_kernel(emb_hbm, cmd_idx_hbm, obs_idx_hbm, cmd_out_hbm, obs_out_hbm,
                idx_v, rows_v, out_v, obs_idx_v, obs_rows_v, obs_out_v, sem):
    wid = lax.axis_index("s") * _NC + lax.axis_index("c")

    pltpu.sync_copy(cmd_idx_hbm.at[wid], idx_v)
    copies = [
        pltpu.async_copy(
            emb_hbm.at[idx_v.at[j]],
            rows_v.at[pl.ds(j * _IDX_CHUNK, _IDX_CHUNK)],
            sem,
        )
        for j in range(_N_CHUNKS)
    ]
    for cp in copies:
      cp.wait()

    def cmd_body(c, carry):
      def tok_body(t, accs):
        r = c * _CMD_LEN + t
        return tuple(
            accs[v] + rows_v[r, pl.ds(v * 16, 16)] for v in range(4)
        )
      zeros = tuple(jnp.zeros((16,), jnp.float32) for _ in range(4))
      accs = lax.fori_loop(0, _CMD_LEN, tok_body, zeros)
      for v in range(4):
        out_v[c, pl.ds(v * 16, 16)] = accs[v]
      return carry

    lax.fori_loop(0, _CMDS_PER_W, cmd_body, 0)
    pltpu.sync_copy(out_v, cmd_out_hbm.at[pl.ds(wid * _CMDS_PER_W, _CMDS_PER_W)])

    @pl.when(wid == 0)
    def _():
      pltpu.sync_copy(obs_idx_hbm, obs_idx_v)
      ocopies = [
          pltpu.async_copy(
              emb_hbm.at[obs_idx_v.at[j]],
              obs_rows_v.at[pl.ds(j * _IDX_CHUNK, _IDX_CHUNK)],
              sem,
          )
          for j in range(2)
      ]
      for cp in ocopies:
        cp.wait()

      def obs_body(t, accs):
        return tuple(
            accs[v] + obs_rows_v[t, pl.ds(v * 16, 16)] for v in range(4)
        )
      zeros = tuple(jnp.zeros((16,), jnp.float32) for _ in range(4))
      accs = lax.fori_loop(0, _OBS_LEN, obs_body, zeros)
      for v in range(4):
        obs_out_v[0, pl.ds(v * 16, 16)] = accs[v]
      pltpu.sync_copy(obs_out_v, obs_out_hbm)

  return sc_kernel


def _tc_epilogue(cmd_sums_ref, obs_sum_ref, cw_ref, cb_ref, aws_ref, awc_ref,
                 ab_ref, g_ref, scores_ref, idx_ref, value_ref):
  obs_mean = obs_sum_ref[...] * (1.0 / _OBS_LEN)
  value_ref[...] = (
      jnp.sum(obs_mean * cw_ref[...], axis=1, keepdims=True) + cb_ref[...]
  )
  s_state = jnp.sum(obs_mean * aws_ref[...], axis=1, keepdims=True) + ab_ref[...]
  cmd_mean = cmd_sums_ref[...] * (1.0 / _CMD_LEN)
  scores = jnp.sum(cmd_mean * awc_ref[...], axis=1, keepdims=True) + s_state
  scores_ref[...] = scores
  z = scores + g_ref[...]
  m = jnp.max(z)
  iota = lax.broadcasted_iota(jnp.int32, (_N_CMDS, 1), 0)
  idx_ref[...] = jnp.min(
      jnp.where(z == m, iota, jnp.int32(2**30)), axis=0, keepdims=True
  )


def kernel(obs, commands, emb_table, critic_w, critic_b, att_w, att_b):
  cmd_idx = commands.reshape(_NW, _N_CHUNKS, _IDX_CHUNK)
  obs_idx = jnp.concatenate(
      [obs, jnp.zeros((_OBS_PAD - _OBS_LEN,), jnp.int32)]
  ).reshape(2, _IDX_CHUNK)

  cmd_sums, obs_sum = _sc_pool_kernel()(emb_table, cmd_idx, obs_idx)

  scores2d, idx2d, value = pl.pallas_call(
      _tc_epilogue,
      out_shape=[
          jax.ShapeDtypeStruct((_N_CMDS, 1), jnp.float32),
          jax.ShapeDtypeStruct((1, 1), jnp.int32),
          jax.ShapeDtypeStruct((1, 1), jnp.float32),
      ],
  )(
      cmd_sums,
      obs_sum,
      critic_w.reshape(1, _H),
      critic_b.reshape(1, 1),
      att_w[:_H].reshape(1, _H),
      att_w[_H:].reshape(1, _H),
      att_b.reshape(1, 1),
      _GUMBEL,
  )
  return scores2d[:, 0], idx2d[0, 0], value
